# baseline (speedup 1.0000x reference)
import functools

import jax
import jax.numpy as jnp
from jax import lax
from jax.experimental import pallas as pl
from jax.experimental.pallas import tpu as pltpu
from jax.experimental.pallas import tpu_sc as plsc

KNN = 40
BN = 256
BM = 32
NEG_SLOPE = 0.2


def _lrelu(z):
  return jnp.where(z >= 0, z, NEG_SLOPE * z)


def _dot(a, b):
  return lax.dot_general(a, b, (((1,), (0,)), ((), ())),
                         preferred_element_type=jnp.float32)


def _dot_nt(a, b):
  return lax.dot_general(a, b, (((1,), (1,)), ((), ())),
                         preferred_element_type=jnp.float32)


def _topk_octets(x_all, sqj, n, b, x_row, idx_ref, sc_ref):
  no = BN // 8
  iota8 = lax.broadcasted_iota(jnp.int32, (8, n), 1)
  kiota = lax.broadcasted_iota(jnp.int32, (BN, KNN), 1)
  oiota = lax.broadcasted_iota(jnp.int32, (8, no), 1)
  big = jnp.float32(jnp.inf)

  cur_pieces = []
  for o in range(no):
    x_i8 = x_row(o)
    sqi8 = jnp.sum(x_i8 * x_i8, axis=1, keepdims=True)
    sc0 = (sqi8 - 2.0 * _dot_nt(x_i8, x_all)) + sqj[None, :]
    sc_ref[pl.ds(o * 8, 8), :] = sc0
    cur_pieces.append(jnp.min(sc0, axis=1, keepdims=True))
  curs0 = jnp.concatenate(cur_pieces, axis=1)

  def mstep(m, carry):
    idx_acc, curs = carry
    pieces = []
    new_curs = curs
    for o in range(no):
      sl = pl.ds(o * 8, 8)
      sc_o = sc_ref[sl, :]
      cur_o = curs[:, o:o + 1]
      idxv = jnp.min(jnp.where(sc_o <= cur_o, iota8, n), axis=1,
                     keepdims=True)
      masked = jnp.where(iota8 == idxv, big, sc_o)
      sc_ref[sl, :] = masked
      nxt = jnp.min(masked, axis=1, keepdims=True)
      new_curs = jnp.where(oiota == o, nxt, new_curs)
      pieces.append(idxv)
    idxall = jnp.concatenate(pieces, axis=0)
    idx_acc = jnp.where(kiota == m, idxall + b * n, idx_acc)
    return (idx_acc, new_curs)

  idx_acc, _ = lax.fori_loop(
      0, KNN, mstep, (jnp.zeros((BN, KNN), jnp.int32), curs0))
  idx_ref[0] = idx_acc


def _knn_body(x_ref, idx_ref, sc_ref, *, n):
  b = pl.program_id(0)
  j = pl.program_id(1)
  x_all = x_ref[0]
  sqj = jnp.sum(x_all * x_all, axis=1)
  x_row = lambda o: x_ref[0, pl.ds(j * BN + o * 8, 8), :]
  _topk_octets(x_all, sqj, n, b, x_row, idx_ref, sc_ref)


def _knn(x_full):
  B, n, cp = x_full.shape
  return pl.pallas_call(
      functools.partial(_knn_body, n=n),
      grid=(B, n // BN),
      in_specs=[pl.BlockSpec((1, n, cp), lambda b, j: (b, 0, 0))],
      out_specs=pl.BlockSpec((1, BN, KNN), lambda b, j: (b, j, 0)),
      out_shape=jax.ShapeDtypeStruct((B, n, KNN), jnp.int32),
      scratch_shapes=[pltpu.VMEM((BN, n), jnp.float32)],
  )(x_full)


def _knn_xform_body(p_ref, f_ref, t8_ref, idx_ref, x8_ref, sc_ref, *, n):
  b = pl.program_id(0)
  j = pl.program_id(1)
  t8 = t8_ref[0]
  x_all = _dot(p_ref[0], t8) + f_ref[0]
  sqj = jnp.sum(x_all * x_all, axis=1)

  def x_row(o):
    sl = pl.ds(j * BN + o * 8, 8)
    return _dot(p_ref[0, sl, :], t8) + f_ref[0, sl, :]

  _topk_octets(x_all, sqj, n, b, x_row, idx_ref, sc_ref)
  sl = pl.ds(j * BN, BN)
  x8_ref[0] = _dot(p_ref[0, sl, :], t8) + f_ref[0, sl, :]


def _knn_xform(p8, f8, t8):
  B, n, cp = p8.shape
  return pl.pallas_call(
      functools.partial(_knn_xform_body, n=n),
      grid=(B, n // BN),
      in_specs=[
          pl.BlockSpec((1, n, cp), lambda b, j: (b, 0, 0)),
          pl.BlockSpec((1, n, cp), lambda b, j: (b, 0, 0)),
          pl.BlockSpec((1, 16, 16), lambda b, j: (b, 0, 0)),
      ],
      out_specs=[
          pl.BlockSpec((1, BN, KNN), lambda b, j: (b, j, 0)),
          pl.BlockSpec((1, BN, 16), lambda b, j: (b, j, 0)),
      ],
      out_shape=[
          jax.ShapeDtypeStruct((B, n, KNN), jnp.int32),
          jax.ShapeDtypeStruct((B, n, 16), jnp.float32),
      ],
      scratch_shapes=[pltpu.VMEM((BN, n), jnp.float32)],
  )(p8, f8, t8)


def _edge_core(xj, x_i, w0a, w0b, b0, w1, b1, c1):
  c = x_i.shape[-1]
  d = xj - x_i[:, None, :]
  ga = _dot(d.reshape(BM * KNN, c), w0a).reshape(BM, KNN, 64)
  gb = _dot(x_i, w0b) + b0
  g = _lrelu(ga + gb[:, None, :])
  z = _lrelu(_dot(g.reshape(BM * KNN, 64), w1) + b1)
  return jnp.max(z.reshape(BM, KNN, c1), axis=1)


def _edge_tnet_body(xj_ref, x_ref, w0a_ref, w0b_ref, b0_ref, w1_ref, b1_ref,
                    out_ref):
  out_ref[0] = _edge_core(xj_ref[0], x_ref[0], w0a_ref[...], w0b_ref[...],
                          b0_ref[...], w1_ref[...], b1_ref[...], 128)


def _edge_tnet(xj, x, w0a, w0b, b0, w1, b1):
  B, n, c = x.shape
  wspec = lambda r, cc: pl.BlockSpec((r, cc), lambda b, j: (0, 0))
  return pl.pallas_call(
      _edge_tnet_body,
      grid=(B, n // BM),
      in_specs=[
          pl.BlockSpec((1, BM, KNN, c), lambda b, j: (b, j, 0, 0)),
          pl.BlockSpec((1, BM, c), lambda b, j: (b, j, 0)),
          wspec(c, 64), wspec(c, 64), wspec(1, 64),
          wspec(64, 128), wspec(1, 128),
      ],
      out_specs=pl.BlockSpec((1, BM, 128), lambda b, j: (b, j, 0)),
      out_shape=jax.ShapeDtypeStruct((B, n, 128), jnp.float32),
  )(xj, x, w0a, w0b, b0.reshape(1, 64), w1, b1.reshape(1, 128))


def _edge_conv_body(xj_ref, x_ref, xr_ref, w0a_ref, w0b_ref, b0_ref,
                    w1_ref, b1_ref, lw0_ref, lb0_ref, lw1_ref, lb1_ref,
                    tw_ref, tb_ref, out_ref):
  xi = _edge_core(xj_ref[0], x_ref[0], w0a_ref[...], w0b_ref[...],
                  b0_ref[...], w1_ref[...], b1_ref[...], 64)
  y = jnp.maximum(_dot(xi, lw0_ref[...]) + lb0_ref[...], 0.0)
  y = _dot(y, lw1_ref[...]) + lb1_ref[...]
  out_ref[0] = _dot(xr_ref[0], tw_ref[...]) + tb_ref[...] + y


def _edge_conv(xj, x, w0a, w0b, b0, w1, b1, lw0, lb0, lw1, lb1, tw, tb):
  B, n, c = x.shape
  wspec = lambda r, cc: pl.BlockSpec((r, cc), lambda b, j: (0, 0))
  return pl.pallas_call(
      _edge_conv_body,
      grid=(B, n // BM),
      in_specs=[
          pl.BlockSpec((1, BM, KNN, c), lambda b, j: (b, j, 0, 0)),
          pl.BlockSpec((1, BM, c), lambda b, j: (b, j, 0)),
          pl.BlockSpec((1, BM, c), lambda b, j: (b, j, 0)),
          wspec(c, 64), wspec(c, 64), wspec(1, 64),
          wspec(64, 64), wspec(1, 64),
          wspec(64, 64), wspec(1, 64),
          wspec(64, 64), wspec(1, 64),
          wspec(c, 64), wspec(1, 64),
      ],
      out_specs=pl.BlockSpec((1, BM, 64), lambda b, j: (b, j, 0)),
      out_shape=jax.ShapeDtypeStruct((B, n, 64), jnp.float32),
  )(xj, x, x, w0a, w0b, b0.reshape(1, 64), w1, b1.reshape(1, 64),
    lw0, lb0.reshape(1, 64), lw1, lb1.reshape(1, 64), tw, tb.reshape(1, 64))


def _tnet_tail_body(x1_ref, w2_ref, b2_ref, w30_ref, b30_ref, w31_ref,
                    b31_ref, w4_ref, b4_ref, out_ref, *, n):
  chunk = 128
  w2 = w2_ref[...]
  b2 = b2_ref[...]

  def step(c, m):
    z = _lrelu(_dot(x1_ref[0, pl.ds(c * chunk, chunk), :], w2) + b2)
    return jnp.maximum(m, jnp.max(z, axis=0, keepdims=True))

  m = lax.fori_loop(0, n // chunk, step,
                    jnp.full((1, 1024), -jnp.inf, dtype=jnp.float32))
  v = _lrelu(_dot(m, w30_ref[...]) + b30_ref[...])
  v = _lrelu(_dot(v, w31_ref[...]) + b31_ref[...])
  t = _dot(v, w4_ref[...]) + b4_ref[...]
  t128 = jnp.concatenate([t, jnp.zeros((1, 112), jnp.float32)], axis=1)
  out_ref[0] = jnp.concatenate(
      [t128, jnp.zeros((7, 128), jnp.float32)], axis=0)


def _tnet_tail(x1, t2_w, t2_b, t3_w0, t3_b0, t3_w1, t3_b1, t4_wp, t4_bp):
  B, n, _ = x1.shape
  wspec = lambda r, c: pl.BlockSpec((r, c), lambda b: (0, 0))
  return pl.pallas_call(
      functools.partial(_tnet_tail_body, n=n),
      grid=(B,),
      in_specs=[
          pl.BlockSpec((1, n, 128), lambda b: (b, 0, 0)),
          wspec(128, 1024), wspec(1, 1024),
          wspec(1024, 512), wspec(1, 512),
          wspec(512, 256), wspec(1, 256),
          wspec(256, 16), wspec(1, 16),
      ],
      out_specs=pl.BlockSpec((1, 8, 128), lambda b: (b, 0, 0)),
      out_shape=jax.ShapeDtypeStruct((B, 8, 128), jnp.float32),
  )(x1, t2_w, t2_b.reshape(1, 1024), t3_w0, t3_b0.reshape(1, 512),
    t3_w1, t3_b1.reshape(1, 256), t4_wp, t4_bp)


_SC_WORKERS = 32
_SC_CHUNK = 128


def _gather_rows(table, idx_flat):
  e_total = idx_flat.shape[0]
  d = table.shape[1]
  per_w = e_total // _SC_WORKERS
  nch = per_w // _SC_CHUNK
  mesh = plsc.VectorSubcoreMesh(core_axis_name="c", subcore_axis_name="s")

  @functools.partial(
      pl.kernel, mesh=mesh,
      out_type=jax.ShapeDtypeStruct((e_total, d), jnp.float32),
      compiler_params=pltpu.CompilerParams(use_tc_tiling_on_sc=False),
      scratch_types=[
          pltpu.VMEM((_SC_CHUNK,), jnp.int32),
          pltpu.VMEM((_SC_CHUNK, d), jnp.float32),
          pltpu.SemaphoreType.DMA,
      ],
  )
  def k(table_hbm, idx_hbm, out_hbm, idx_v, rows_v, sem):
    wid = lax.axis_index("s") * 2 + lax.axis_index("c")
    base = wid * per_w

    def chunk(i, carry):
      off = base + i * _SC_CHUNK
      pltpu.sync_copy(idx_hbm.at[pl.ds(off, _SC_CHUNK)], idx_v)
      pltpu.async_copy(table_hbm.at[idx_v], rows_v, sem).wait()
      pltpu.sync_copy(rows_v, out_hbm.at[pl.ds(off, _SC_CHUNK)])
      return carry

    lax.fori_loop(0, nch, chunk, 0)

  return k(table, idx_flat)


def _pad_cols(w, rows):
  return jnp.zeros((rows, w.shape[1]), w.dtype).at[: w.shape[0]].set(w)


def kernel(positions, features, t1_w0, t1_b0, t1_w1, t1_b1, t2_w, t2_b,
           t3_w0, t3_b0, t3_w1, t3_b1, t4_w, t4_b,
           c0_w0, c0_b0, c0_w1, c0_b1, c1_w0, c1_b0, c1_w1, c1_b1,
           l0_w0, l0_b0, l0_w1, l0_b1, l1_w0, l1_b0, l1_w1, l1_b1,
           lt0_w, lt0_b, lt1_w, lt1_b):
  B, n, _ = positions.shape
  p16 = jnp.concatenate(
      [positions, jnp.zeros((B, n, 13), jnp.float32)], axis=-1)
  f16 = jnp.concatenate(
      [jnp.zeros((B, n, 3), jnp.float32), features,
       jnp.zeros((B, n, 10), jnp.float32)], axis=-1)

  idx0 = _knn(p16)
  xj0 = _gather_rows(p16.reshape(B * n, 16), idx0.reshape(-1))
  x1 = _edge_tnet(xj0.reshape(B, n, KNN, 16), p16,
                  _pad_cols(t1_w0[:3], 16), _pad_cols(t1_w0[3:], 16), t1_b0,
                  t1_w1, t1_b1)
  t4_wp = _pad_cols(t4_w.T, 16).T
  t4_bp = jnp.zeros((1, 16), jnp.float32).at[0, :9].set(t4_b)
  tpad = _tnet_tail(x1, t2_w, t2_b, t3_w0, t3_b0, t3_w1, t3_b1, t4_wp, t4_bp)
  t16 = jnp.zeros((B, 16, 16), jnp.float32).at[:, :3, :3].set(
      tpad[:, 0, :9].reshape(B, 3, 3))

  idx1, xcat16 = _knn_xform(p16, f16, t16)
  xj1 = _gather_rows(xcat16.reshape(B * n, 16), idx1.reshape(-1))
  x_c0 = _edge_conv(
      xj1.reshape(B, n, KNN, 16), xcat16,
      _pad_cols(c0_w0[:6], 16), _pad_cols(c0_w0[6:], 16), c0_b0,
      c0_w1, c0_b1, l0_w0, l0_b0, l0_w1, l0_b1,
      _pad_cols(lt0_w, 16), lt0_b)

  idx2 = _knn(x_c0)
  xj2 = _gather_rows(x_c0.reshape(B * n, 64), idx2.reshape(-1))
  out = _edge_conv(
      xj2.reshape(B, n, KNN, 64), x_c0,
      c1_w0[:64], c1_w0[64:], c1_b0,
      c1_w1, c1_b1, l1_w0, l1_b0, l1_w1, l1_b1, lt1_w, lt1_b)
  return out


# speedup vs baseline: 3.0574x; 3.0574x over previous
import functools

import jax
import jax.numpy as jnp
from jax import lax
from jax.experimental import pallas as pl
from jax.experimental.pallas import tpu as pltpu
from jax.experimental.pallas import tpu_sc as plsc

KNN = 40
BN = 256
BM = 32
NEG_SLOPE = 0.2


def _lrelu(z):
  return jnp.where(z >= 0, z, NEG_SLOPE * z)


def _dot(a, b):
  return lax.dot_general(a, b, (((1,), (0,)), ((), ())),
                         preferred_element_type=jnp.float32)


def _dot_nt(a, b):
  return lax.dot_general(a, b, (((1,), (1,)), ((), ())),
                         preferred_element_type=jnp.float32)


def _topk_octets(x_all, sqj, n, b, x_row, idx_ref, sc_ref):
  iota8 = lax.broadcasted_iota(jnp.int32, (8, n), 1)
  kiota = lax.broadcasted_iota(jnp.int32, (BN, KNN), 1)
  big = jnp.float32(jnp.inf)

  for o in range(BN // 8):
    x_i8 = x_row(o)
    sqi8 = jnp.sum(x_i8 * x_i8, axis=1, keepdims=True)
    sc_ref[pl.ds(o * 8, 8), :] = (
        (sqi8 - 2.0 * _dot_nt(x_i8, x_all)) + sqj[None, :])

  def mstep(m, idx_acc):
    pieces = []
    for o in range(BN // 8):
      sl = pl.ds(o * 8, 8)
      sc_o = sc_ref[sl, :]
      cur = jnp.min(sc_o, axis=1, keepdims=True)
      idxv = jnp.min(jnp.where(sc_o <= cur, iota8, n), axis=1, keepdims=True)
      sc_ref[sl, :] = jnp.where(iota8 == idxv, big, sc_o)
      pieces.append(idxv)
    idxall = jnp.concatenate(pieces, axis=0)
    return jnp.where(kiota == m, idxall + b * n, idx_acc)

  idx_acc = lax.fori_loop(0, KNN, mstep, jnp.zeros((BN, KNN), jnp.int32))
  idx_ref[0] = idx_acc


def _knn_body(x_ref, idx_ref, sc_ref, *, n):
  b = pl.program_id(0)
  j = pl.program_id(1)
  x_all = x_ref[0]
  sqj = jnp.sum(x_all * x_all, axis=1)
  x_row = lambda o: x_ref[0, pl.ds(j * BN + o * 8, 8), :]
  _topk_octets(x_all, sqj, n, b, x_row, idx_ref, sc_ref)


def _knn(x_full):
  B, n, cp = x_full.shape
  return pl.pallas_call(
      functools.partial(_knn_body, n=n),
      grid=(B, n // BN),
      in_specs=[pl.BlockSpec((1, n, cp), lambda b, j: (b, 0, 0))],
      out_specs=pl.BlockSpec((1, BN, KNN), lambda b, j: (b, j, 0)),
      out_shape=jax.ShapeDtypeStruct((B, n, KNN), jnp.int32),
      scratch_shapes=[pltpu.VMEM((BN, n), jnp.float32)],
  )(x_full)


def _knn_xform_body(p_ref, f_ref, t8_ref, idx_ref, x8_ref, sc_ref, *, n):
  b = pl.program_id(0)
  j = pl.program_id(1)
  t8 = t8_ref[0]
  x_all = _dot(p_ref[0], t8) + f_ref[0]
  sqj = jnp.sum(x_all * x_all, axis=1)

  def x_row(o):
    sl = pl.ds(j * BN + o * 8, 8)
    return _dot(p_ref[0, sl, :], t8) + f_ref[0, sl, :]

  _topk_octets(x_all, sqj, n, b, x_row, idx_ref, sc_ref)
  sl = pl.ds(j * BN, BN)
  x8_ref[0] = _dot(p_ref[0, sl, :], t8) + f_ref[0, sl, :]


def _knn_xform(p8, f8, t8):
  B, n, cp = p8.shape
  return pl.pallas_call(
      functools.partial(_knn_xform_body, n=n),
      grid=(B, n // BN),
      in_specs=[
          pl.BlockSpec((1, n, cp), lambda b, j: (b, 0, 0)),
          pl.BlockSpec((1, n, cp), lambda b, j: (b, 0, 0)),
          pl.BlockSpec((1, 16, 16), lambda b, j: (b, 0, 0)),
      ],
      out_specs=[
          pl.BlockSpec((1, BN, KNN), lambda b, j: (b, j, 0)),
          pl.BlockSpec((1, BN, 16), lambda b, j: (b, j, 0)),
      ],
      out_shape=[
          jax.ShapeDtypeStruct((B, n, KNN), jnp.int32),
          jax.ShapeDtypeStruct((B, n, 16), jnp.float32),
      ],
      scratch_shapes=[pltpu.VMEM((BN, n), jnp.float32)],
  )(p8, f8, t8)


def _edge_core(xj, x_i, w0a, w0b, b0, w1, b1, c1):
  c = x_i.shape[-1]
  d = xj - x_i[:, None, :]
  ga = _dot(d.reshape(BM * KNN, c), w0a).reshape(BM, KNN, 64)
  gb = _dot(x_i, w0b) + b0
  g = _lrelu(ga + gb[:, None, :])
  z = _lrelu(_dot(g.reshape(BM * KNN, 64), w1) + b1)
  return jnp.max(z.reshape(BM, KNN, c1), axis=1)


def _edge_tnet_body(xj_ref, x_ref, w0a_ref, w0b_ref, b0_ref, w1_ref, b1_ref,
                    out_ref):
  out_ref[0] = _edge_core(xj_ref[0], x_ref[0], w0a_ref[...], w0b_ref[...],
                          b0_ref[...], w1_ref[...], b1_ref[...], 128)


def _edge_tnet(xj, x, w0a, w0b, b0, w1, b1):
  B, n, c = x.shape
  wspec = lambda r, cc: pl.BlockSpec((r, cc), lambda b, j: (0, 0))
  return pl.pallas_call(
      _edge_tnet_body,
      grid=(B, n // BM),
      in_specs=[
          pl.BlockSpec((1, BM, KNN, c), lambda b, j: (b, j, 0, 0)),
          pl.BlockSpec((1, BM, c), lambda b, j: (b, j, 0)),
          wspec(c, 64), wspec(c, 64), wspec(1, 64),
          wspec(64, 128), wspec(1, 128),
      ],
      out_specs=pl.BlockSpec((1, BM, 128), lambda b, j: (b, j, 0)),
      out_shape=jax.ShapeDtypeStruct((B, n, 128), jnp.float32),
  )(xj, x, w0a, w0b, b0.reshape(1, 64), w1, b1.reshape(1, 128))


def _edge_conv_body(xj_ref, x_ref, xr_ref, w0a_ref, w0b_ref, b0_ref,
                    w1_ref, b1_ref, lw0_ref, lb0_ref, lw1_ref, lb1_ref,
                    tw_ref, tb_ref, out_ref):
  xi = _edge_core(xj_ref[0], x_ref[0], w0a_ref[...], w0b_ref[...],
                  b0_ref[...], w1_ref[...], b1_ref[...], 64)
  y = jnp.maximum(_dot(xi, lw0_ref[...]) + lb0_ref[...], 0.0)
  y = _dot(y, lw1_ref[...]) + lb1_ref[...]
  out_ref[0] = _dot(xr_ref[0], tw_ref[...]) + tb_ref[...] + y


def _edge_conv(xj, x, w0a, w0b, b0, w1, b1, lw0, lb0, lw1, lb1, tw, tb):
  B, n, c = x.shape
  wspec = lambda r, cc: pl.BlockSpec((r, cc), lambda b, j: (0, 0))
  return pl.pallas_call(
      _edge_conv_body,
      grid=(B, n // BM),
      in_specs=[
          pl.BlockSpec((1, BM, KNN, c), lambda b, j: (b, j, 0, 0)),
          pl.BlockSpec((1, BM, c), lambda b, j: (b, j, 0)),
          pl.BlockSpec((1, BM, c), lambda b, j: (b, j, 0)),
          wspec(c, 64), wspec(c, 64), wspec(1, 64),
          wspec(64, 64), wspec(1, 64),
          wspec(64, 64), wspec(1, 64),
          wspec(64, 64), wspec(1, 64),
          wspec(c, 64), wspec(1, 64),
      ],
      out_specs=pl.BlockSpec((1, BM, 64), lambda b, j: (b, j, 0)),
      out_shape=jax.ShapeDtypeStruct((B, n, 64), jnp.float32),
  )(xj, x, x, w0a, w0b, b0.reshape(1, 64), w1, b1.reshape(1, 64),
    lw0, lb0.reshape(1, 64), lw1, lb1.reshape(1, 64), tw, tb.reshape(1, 64))


def _tnet_tail_body(x1_ref, w2_ref, b2_ref, w30_ref, b30_ref, w31_ref,
                    b31_ref, w4_ref, b4_ref, out_ref, *, n):
  chunk = 128
  w2 = w2_ref[...]
  b2 = b2_ref[...]

  def step(c, m):
    z = _lrelu(_dot(x1_ref[0, pl.ds(c * chunk, chunk), :], w2) + b2)
    return jnp.maximum(m, jnp.max(z, axis=0, keepdims=True))

  m = lax.fori_loop(0, n // chunk, step,
                    jnp.full((1, 1024), -jnp.inf, dtype=jnp.float32))
  v = _lrelu(_dot(m, w30_ref[...]) + b30_ref[...])
  v = _lrelu(_dot(v, w31_ref[...]) + b31_ref[...])
  t = _dot(v, w4_ref[...]) + b4_ref[...]
  t128 = jnp.concatenate([t, jnp.zeros((1, 112), jnp.float32)], axis=1)
  out_ref[0] = jnp.concatenate(
      [t128, jnp.zeros((7, 128), jnp.float32)], axis=0)


def _tnet_tail(x1, t2_w, t2_b, t3_w0, t3_b0, t3_w1, t3_b1, t4_wp, t4_bp):
  B, n, _ = x1.shape
  wspec = lambda r, c: pl.BlockSpec((r, c), lambda b: (0, 0))
  return pl.pallas_call(
      functools.partial(_tnet_tail_body, n=n),
      grid=(B,),
      in_specs=[
          pl.BlockSpec((1, n, 128), lambda b: (b, 0, 0)),
          wspec(128, 1024), wspec(1, 1024),
          wspec(1024, 512), wspec(1, 512),
          wspec(512, 256), wspec(1, 256),
          wspec(256, 16), wspec(1, 16),
      ],
      out_specs=pl.BlockSpec((1, 8, 128), lambda b: (b, 0, 0)),
      out_shape=jax.ShapeDtypeStruct((B, 8, 128), jnp.float32),
  )(x1, t2_w, t2_b.reshape(1, 1024), t3_w0, t3_b0.reshape(1, 512),
    t3_w1, t3_b1.reshape(1, 256), t4_wp, t4_bp)


_SC_WORKERS = 32
_SC_CHUNK = 128


def _gather_rows(table, idx_flat):
  e_total = idx_flat.shape[0]
  d = table.shape[1]
  per_w = e_total // _SC_WORKERS
  nch = per_w // _SC_CHUNK
  mesh = plsc.VectorSubcoreMesh(core_axis_name="c", subcore_axis_name="s")

  @functools.partial(
      pl.kernel, mesh=mesh,
      out_type=jax.ShapeDtypeStruct((e_total, d), jnp.float32),
      compiler_params=pltpu.CompilerParams(use_tc_tiling_on_sc=False),
      scratch_types=[
          pltpu.VMEM((_SC_CHUNK,), jnp.int32),
          pltpu.VMEM((_SC_CHUNK, d), jnp.float32),
          pltpu.SemaphoreType.DMA,
      ],
  )
  def k(table_hbm, idx_hbm, out_hbm, idx_v, rows_v, sem):
    wid = lax.axis_index("s") * 2 + lax.axis_index("c")
    base = wid * per_w

    def chunk(i, carry):
      off = base + i * _SC_CHUNK
      pltpu.sync_copy(idx_hbm.at[pl.ds(off, _SC_CHUNK)], idx_v)
      pltpu.async_copy(table_hbm.at[idx_v], rows_v, sem).wait()
      pltpu.sync_copy(rows_v, out_hbm.at[pl.ds(off, _SC_CHUNK)])
      return carry

    lax.fori_loop(0, nch, chunk, 0)

  return k(table, idx_flat)


def _pad_cols(w, rows):
  return jnp.zeros((rows, w.shape[1]), w.dtype).at[: w.shape[0]].set(w)


def kernel(positions, features, t1_w0, t1_b0, t1_w1, t1_b1, t2_w, t2_b,
           t3_w0, t3_b0, t3_w1, t3_b1, t4_w, t4_b,
           c0_w0, c0_b0, c0_w1, c0_b1, c1_w0, c1_b0, c1_w1, c1_b1,
           l0_w0, l0_b0, l0_w1, l0_b1, l1_w0, l1_b0, l1_w1, l1_b1,
           lt0_w, lt0_b, lt1_w, lt1_b):
  B, n, _ = positions.shape
  p16 = jnp.concatenate(
      [positions, jnp.zeros((B, n, 13), jnp.float32)], axis=-1)
  f16 = jnp.concatenate(
      [jnp.zeros((B, n, 3), jnp.float32), features,
       jnp.zeros((B, n, 10), jnp.float32)], axis=-1)

  idx0 = _knn(p16)
  xj0 = _gather_rows(p16.reshape(B * n, 16), idx0.reshape(-1))
  x1 = _edge_tnet(xj0.reshape(B, n, KNN, 16), p16,
                  _pad_cols(t1_w0[:3], 16), _pad_cols(t1_w0[3:], 16), t1_b0,
                  t1_w1, t1_b1)
  t4_wp = _pad_cols(t4_w.T, 16).T
  t4_bp = jnp.zeros((1, 16), jnp.float32).at[0, :9].set(t4_b)
  tpad = _tnet_tail(x1, t2_w, t2_b, t3_w0, t3_b0, t3_w1, t3_b1, t4_wp, t4_bp)
  t16 = jnp.zeros((B, 16, 16), jnp.float32).at[:, :3, :3].set(
      tpad[:, 0, :9].reshape(B, 3, 3))

  idx1, xcat16 = _knn_xform(p16, f16, t16)
  xj1 = _gather_rows(xcat16.reshape(B * n, 16), idx1.reshape(-1))
  x_c0 = _edge_conv(
      xj1.reshape(B, n, KNN, 16), xcat16,
      _pad_cols(c0_w0[:6], 16), _pad_cols(c0_w0[6:], 16), c0_b0,
      c0_w1, c0_b1, l0_w0, l0_b0, l0_w1, l0_b1,
      _pad_cols(lt0_w, 16), lt0_b)

  idx2 = _knn(x_c0)
  xj2 = _gather_rows(x_c0.reshape(B * n, 64), idx2.reshape(-1))
  out = _edge_conv(
      xj2.reshape(B, n, KNN, 64), x_c0,
      c1_w0[:64], c1_w0[64:], c1_b0,
      c1_w1, c1_b1, l1_w0, l1_b0, l1_w1, l1_b1, lt1_w, lt1_b)
  return out
